# baseline (speedup 1.0000x reference)
import functools

import jax
import jax.numpy as jnp
from jax import lax
from jax.experimental import pallas as pl
from jax.experimental.pallas import tpu as pltpu
from jax.experimental.pallas import tpu_sc as plsc

N = 50000
D = 128
NUM_REL = 8
H1 = 64
H2 = 32
LW = 128

NW = 32
PER_W = 1568
NP = NW * PER_W
NCH = 7
CH = PER_W // NCH


def _tc_slab_body(h, x_ref, w_ref, b_ref, t_ref, s_ref):
    k = LW // h
    y = jax.lax.dot_general(
        x_ref[...], w_ref[...],
        (((1,), (0,)), ((), ())),
        preferred_element_type=jnp.float32) + b_ref[...]
    s_ref[...] = y[:, :h]
    for r in range(NUM_REL):
        p, o = divmod(r, k)
        t_ref[p, :, o * h:(o + 1) * h] = y[:, (r + 1) * h:(r + 2) * h]


def _tc_slab(x, wbig, bbig, h, blk):
    n_rows = x.shape[0]
    d_in, d_out = wbig.shape
    npl = NUM_REL // (LW // h)
    nb = n_rows // blk
    return pl.pallas_call(
        functools.partial(_tc_slab_body, h),
        grid=(nb,),
        in_specs=[
            pl.BlockSpec((blk, d_in), lambda i: (i, 0)),
            pl.BlockSpec((d_in, d_out), lambda i: (0, 0)),
            pl.BlockSpec((1, d_out), lambda i: (0, 0)),
        ],
        out_specs=[
            pl.BlockSpec((npl, blk, LW), lambda i: (0, i, 0)),
            pl.BlockSpec((blk, h), lambda i: (i, 0)),
        ],
        out_shape=[
            jax.ShapeDtypeStruct((npl, n_rows, LW), jnp.float32),
            jax.ShapeDtypeStruct((n_rows, h), jnp.float32),
        ],
    )(x, wbig, bbig)


def _make_sc_gather(h):
    grp = h // 16
    k = LW // h
    unr = 8
    mesh = plsc.VectorSubcoreMesh(core_axis_name="c", subcore_axis_name="s")

    @functools.partial(
        pl.kernel,
        out_type=jax.ShapeDtypeStruct((NP, h), jnp.float32),
        mesh=mesh,
        scratch_types=[
            [pltpu.VMEM((CH,), jnp.int32) for _ in range(NUM_REL)],
            pltpu.VMEM((CH, h), jnp.float32),
            pltpu.VMEM((CH, LW), jnp.float32),
            pltpu.VMEM((CH, LW), jnp.float32),
            pltpu.SemaphoreType.DMA,
            pltpu.SemaphoreType.DMA,
            pltpu.SemaphoreType.DMA,
            pltpu.SemaphoreType.DMA,
        ],
    )
    def sc_fn(tab_hbm, self_hbm, idx_hbm, out_hbm, idx_v, acc_v, g0, g1,
              s_acc, s0, s1, s_idx):
        wid = lax.axis_index("s") * 2 + lax.axis_index("c")
        base0 = wid * PER_W
        bufs = (g0, g1)
        sems = (s0, s1)

        def chunk_body(c, carry):
            base = base0 + c * CH
            idx_cps = [
                pltpu.async_copy(
                    idx_hbm.at[pl.ds(r * NP + base, CH)], idx_v[r], s_idx)
                for r in range(NUM_REL)
            ]
            for cp in idx_cps:
                cp.wait()
            acc_cp = pltpu.async_copy(
                self_hbm.at[pl.ds(base, CH)], acc_v, s_acc)
            handles = [
                pltpu.async_copy(tab_hbm.at[0].at[idx_v[0]], g0, s0),
                pltpu.async_copy(tab_hbm.at[1 // k].at[idx_v[1]], g1, s1),
            ]
            acc_cp.wait()
            for r in range(NUM_REL):
                p = r % 2
                buf = bufs[p]
                handles[p].wait()
                last = r == NUM_REL - 1
                off = (r % k) * h

                def acc_body(i, _, buf=buf, last=last, off=off):
                    row = i * unr
                    for u in range(unr):
                        for j in range(grp):
                            dst = pl.ds(j * 16, 16)
                            src = pl.ds(off + j * 16, 16)
                            v = acc_v[row + u, dst] + buf[row + u, src]
                            if last:
                                acc_v[row + u, dst] = jnp.maximum(v, 0.0)
                            else:
                                acc_v[row + u, dst] = v
                    return 0

                lax.fori_loop(0, CH // unr, acc_body, 0)
                if r + 2 < NUM_REL:
                    handles[p] = pltpu.async_copy(
                        tab_hbm.at[(r + 2) // k].at[idx_v[r + 2]], buf,
                        sems[p])
            pltpu.sync_copy(acc_v, out_hbm.at[pl.ds(base, CH)])
            return carry

        lax.fori_loop(0, NCH, chunk_body, 0)

    return sc_fn


_sc_gather_h1 = _make_sc_gather(H1)
_sc_gather_h2 = _make_sc_gather(H2)


def _tc_final_body(h2_ref, wf_ref, bf_ref, o_ref, acc_ref):
    i = pl.program_id(0)

    @pl.when(i == 0)
    def _():
        acc_ref[...] = jnp.zeros_like(acc_ref)

    acc_ref[...] += jnp.sum(h2_ref[...], axis=0, keepdims=True)

    @pl.when(i == pl.num_programs(0) - 1)
    def _():
        mean = acc_ref[...] * (1.0 / N)
        o_ref[...] = jax.lax.dot_general(
            mean, wf_ref[...], (((1,), (1,)), ((), ())),
            preferred_element_type=jnp.float32) + bf_ref[...]


def _tc_final(h2, wf, bf):
    blk = 2000
    return pl.pallas_call(
        _tc_final_body,
        grid=(N // blk,),
        in_specs=[
            pl.BlockSpec((blk, H2), lambda i: (i, 0)),
            pl.BlockSpec((D, H2), lambda i: (0, 0)),
            pl.BlockSpec((1, D), lambda i: (0, 0)),
        ],
        out_specs=pl.BlockSpec((1, D), lambda i: (0, 0)),
        out_shape=jax.ShapeDtypeStruct((1, D), jnp.float32),
        scratch_shapes=[pltpu.VMEM((1, H2), jnp.float32)],
    )(h2, wf, bf[None])


def kernel(node_features, adjacency_list, W1, b1, R1, W2, b2, R2, Wf, bf):
    pad = NP - N
    feats = jnp.pad(node_features, ((0, pad), (0, 0)))
    pad_idx = jnp.broadcast_to(
        jnp.arange(N, NP, dtype=jnp.int32), (NUM_REL, pad))
    adjp = jnp.concatenate([adjacency_list, pad_idx], axis=1)
    idx = adjp.reshape(NUM_REL * NP)

    wbig1 = jnp.concatenate([W1.T] + [R1[r].T for r in range(NUM_REL)], axis=1)
    bbig1 = jnp.concatenate(
        [b1, jnp.zeros((NUM_REL * H1,), jnp.float32)])[None]
    wbig2 = jnp.concatenate([W2.T] + [R2[r].T for r in range(NUM_REL)], axis=1)
    bbig2 = jnp.concatenate(
        [b2, jnp.zeros((NUM_REL * H2,), jnp.float32)])[None]

    t1, s1 = _tc_slab(feats, wbig1, bbig1, H1, blk=1792)
    h1 = _sc_gather_h1(t1, s1, idx)
    t2, s2 = _tc_slab(h1, wbig2, bbig2, H2, blk=1792)
    h2 = _sc_gather_h2(t2, s2, idx)
    out = _tc_final(h2, Wf, bf)
    return out[0]


# speedup vs baseline: 1.0389x; 1.0389x over previous
import functools

import jax
import jax.numpy as jnp
from jax import lax
from jax.experimental import pallas as pl
from jax.experimental.pallas import tpu as pltpu
from jax.experimental.pallas import tpu_sc as plsc

N = 50000
D = 128
NUM_REL = 8
H1 = 64
H2 = 32
LW = 128

NW = 32
PER_W = 1568
NP = NW * PER_W
NCH = 7
CH = PER_W // NCH


def _tc_slab_body(h, x_ref, w_ref, b_ref, t_ref, s_ref):
    k = LW // h
    y = jax.lax.dot_general(
        x_ref[...], w_ref[...],
        (((1,), (0,)), ((), ())),
        preferred_element_type=jnp.float32) + b_ref[...]
    s_ref[...] = y[:, :h]
    for r in range(NUM_REL):
        p, o = divmod(r, k)
        t_ref[p, :, o * h:(o + 1) * h] = y[:, (r + 1) * h:(r + 2) * h]


def _tc_slab(x, wbig, bbig, h, blk):
    n_rows = x.shape[0]
    d_in, d_out = wbig.shape
    npl = NUM_REL // (LW // h)
    nb = n_rows // blk
    return pl.pallas_call(
        functools.partial(_tc_slab_body, h),
        grid=(nb,),
        in_specs=[
            pl.BlockSpec((blk, d_in), lambda i: (i, 0)),
            pl.BlockSpec((d_in, d_out), lambda i: (0, 0)),
            pl.BlockSpec((1, d_out), lambda i: (0, 0)),
        ],
        out_specs=[
            pl.BlockSpec((npl, blk, LW), lambda i: (0, i, 0)),
            pl.BlockSpec((blk, h), lambda i: (i, 0)),
        ],
        out_shape=[
            jax.ShapeDtypeStruct((npl, n_rows, LW), jnp.float32),
            jax.ShapeDtypeStruct((n_rows, h), jnp.float32),
        ],
    )(x, wbig, bbig)


def _make_sc_gather(h):
    grp = h // 16
    k = LW // h
    unr = 8
    mesh = plsc.VectorSubcoreMesh(core_axis_name="c", subcore_axis_name="s")

    @functools.partial(
        pl.kernel,
        out_type=jax.ShapeDtypeStruct((NP, h), jnp.float32),
        mesh=mesh,
        scratch_types=[
            [pltpu.VMEM((CH,), jnp.int32) for _ in range(NUM_REL)],
            pltpu.VMEM((CH, h), jnp.float32),
            pltpu.VMEM((CH, LW), jnp.float32),
            pltpu.VMEM((CH, LW), jnp.float32),
            pltpu.SemaphoreType.DMA,
            pltpu.SemaphoreType.DMA,
            pltpu.SemaphoreType.DMA,
            pltpu.SemaphoreType.DMA,
            pltpu.SemaphoreType.DMA,
        ],
    )
    def sc_fn(tab_hbm, self_hbm, idx_hbm, out_hbm, idx_v, acc_v, g0, g1,
              s_acc, s0, s1, s_idx, s_out):
        wid = lax.axis_index("s") * 2 + lax.axis_index("c")
        base0 = wid * PER_W
        bufs = (g0, g1)
        sems = (s0, s1)

        def load_idx(base):
            for r in range(NUM_REL):
                pltpu.async_copy(
                    idx_hbm.at[pl.ds(r * NP + base, CH)], idx_v[r], s_idx)

        load_idx(base0)

        def wait_idx():
            for r in range(NUM_REL):
                pltpu.make_async_copy(
                    idx_hbm.at[pl.ds(0, CH)], idx_v[r], s_idx).wait()

        def chunk_body(c, carry):
            base = base0 + c * CH
            wait_idx()
            handles = [
                pltpu.async_copy(tab_hbm.at[0].at[idx_v[0]], g0, s0),
                pltpu.async_copy(tab_hbm.at[1 // k].at[idx_v[1]], g1, s1),
            ]

            @pl.when(c != 0)
            def _():
                pltpu.make_async_copy(
                    acc_v, out_hbm.at[pl.ds(base, CH)], s_out).wait()
            acc_cp = pltpu.async_copy(
                self_hbm.at[pl.ds(base, CH)], acc_v, s_acc)
            acc_cp.wait()
            cnext = jnp.where(c + 1 == NCH, 0, c + 1)
            for r in range(NUM_REL):
                p = r % 2
                buf = bufs[p]
                handles[p].wait()
                last = r == NUM_REL - 1
                off = (r % k) * h
                if last:
                    load_idx(base0 + cnext * CH)

                def acc_body(i, _, buf=buf, last=last, off=off):
                    row = i * unr
                    for u in range(unr):
                        for j in range(grp):
                            dst = pl.ds(j * 16, 16)
                            src = pl.ds(off + j * 16, 16)
                            v = acc_v[row + u, dst] + buf[row + u, src]
                            if last:
                                acc_v[row + u, dst] = jnp.maximum(v, 0.0)
                            else:
                                acc_v[row + u, dst] = v
                    return 0

                lax.fori_loop(0, CH // unr, acc_body, 0)
                if r + 2 < NUM_REL:
                    handles[p] = pltpu.async_copy(
                        tab_hbm.at[(r + 2) // k].at[idx_v[r + 2]], buf,
                        sems[p])
            pltpu.async_copy(acc_v, out_hbm.at[pl.ds(base, CH)], s_out)
            return carry

        lax.fori_loop(0, NCH, chunk_body, 0)
        wait_idx()
        pltpu.make_async_copy(
            acc_v, out_hbm.at[pl.ds(base0, CH)], s_out).wait()

    return sc_fn


_sc_gather_h1 = _make_sc_gather(H1)
_sc_gather_h2 = _make_sc_gather(H2)


def _tc_final_body(h2_ref, wf_ref, bf_ref, o_ref, acc_ref):
    i = pl.program_id(0)

    @pl.when(i == 0)
    def _():
        acc_ref[...] = jnp.zeros_like(acc_ref)

    acc_ref[...] += jnp.sum(h2_ref[...], axis=0, keepdims=True)

    @pl.when(i == pl.num_programs(0) - 1)
    def _():
        mean = acc_ref[...] * (1.0 / N)
        o_ref[...] = jax.lax.dot_general(
            mean, wf_ref[...], (((1,), (1,)), ((), ())),
            preferred_element_type=jnp.float32) + bf_ref[...]


def _tc_final(h2, wf, bf):
    blk = 2000
    return pl.pallas_call(
        _tc_final_body,
        grid=(N // blk,),
        in_specs=[
            pl.BlockSpec((blk, H2), lambda i: (i, 0)),
            pl.BlockSpec((D, H2), lambda i: (0, 0)),
            pl.BlockSpec((1, D), lambda i: (0, 0)),
        ],
        out_specs=pl.BlockSpec((1, D), lambda i: (0, 0)),
        out_shape=jax.ShapeDtypeStruct((1, D), jnp.float32),
        scratch_shapes=[pltpu.VMEM((1, H2), jnp.float32)],
    )(h2, wf, bf[None])


def kernel(node_features, adjacency_list, W1, b1, R1, W2, b2, R2, Wf, bf):
    pad = NP - N
    feats = jnp.pad(node_features, ((0, pad), (0, 0)))
    pad_idx = jnp.broadcast_to(
        jnp.arange(N, NP, dtype=jnp.int32), (NUM_REL, pad))
    adjp = jnp.concatenate([adjacency_list, pad_idx], axis=1)
    idx = adjp.reshape(NUM_REL * NP)

    wbig1 = jnp.concatenate([W1.T] + [R1[r].T for r in range(NUM_REL)], axis=1)
    bbig1 = jnp.concatenate(
        [b1, jnp.zeros((NUM_REL * H1,), jnp.float32)])[None]
    wbig2 = jnp.concatenate([W2.T] + [R2[r].T for r in range(NUM_REL)], axis=1)
    bbig2 = jnp.concatenate(
        [b2, jnp.zeros((NUM_REL * H2,), jnp.float32)])[None]

    t1, s1 = _tc_slab(feats, wbig1, bbig1, H1, blk=1792)
    h1 = _sc_gather_h1(t1, s1, idx)
    t2, s2 = _tc_slab(h1, wbig2, bbig2, H2, blk=1792)
    h2 = _sc_gather_h2(t2, s2, idx)
    out = _tc_final(h2, Wf, bf)
    return out[0]


# speedup vs baseline: 1.0568x; 1.0172x over previous
import functools

import jax
import jax.numpy as jnp
from jax import lax
from jax.experimental import pallas as pl
from jax.experimental.pallas import tpu as pltpu
from jax.experimental.pallas import tpu_sc as plsc

N = 50000
D = 128
NUM_REL = 8
H1 = 64
H2 = 32
LW = 128

NW = 32
PER_W = 1568
NP = NW * PER_W
NCH = 7
CH = PER_W // NCH


def _tc_slab_body(h, x_ref, w_ref, b_ref, t_ref, s_ref):
    k = LW // h
    y = jax.lax.dot_general(
        x_ref[...], w_ref[...],
        (((1,), (0,)), ((), ())),
        preferred_element_type=jnp.float32) + b_ref[...]
    s_ref[...] = y[:, :h]
    for r in range(NUM_REL):
        p, o = divmod(r, k)
        t_ref[p, :, o * h:(o + 1) * h] = y[:, (r + 1) * h:(r + 2) * h]


def _tc_slab(x, wbig, bbig, h, blk):
    n_rows = x.shape[0]
    d_in, d_out = wbig.shape
    npl = NUM_REL // (LW // h)
    nb = n_rows // blk
    return pl.pallas_call(
        functools.partial(_tc_slab_body, h),
        grid=(nb,),
        in_specs=[
            pl.BlockSpec((blk, d_in), lambda i: (i, 0)),
            pl.BlockSpec((d_in, d_out), lambda i: (0, 0)),
            pl.BlockSpec((1, d_out), lambda i: (0, 0)),
        ],
        out_specs=[
            pl.BlockSpec((npl, blk, LW), lambda i: (0, i, 0)),
            pl.BlockSpec((blk, h), lambda i: (i, 0)),
        ],
        out_shape=[
            jax.ShapeDtypeStruct((npl, n_rows, LW), jnp.float32),
            jax.ShapeDtypeStruct((n_rows, h), jnp.float32),
        ],
    )(x, wbig, bbig)


def _make_sc_gather(h, fuse_mean=False):
    grp = h // 16
    k = LW // h
    unr = 8
    mesh = plsc.VectorSubcoreMesh(core_axis_name="c", subcore_axis_name="s")
    out_shape = (NW, h) if fuse_mean else (NP, h)

    @functools.partial(
        pl.kernel,
        out_type=jax.ShapeDtypeStruct(out_shape, jnp.float32),
        mesh=mesh,
        scratch_types=[
            [pltpu.VMEM((CH,), jnp.int32) for _ in range(NUM_REL)],
            pltpu.VMEM((CH, h), jnp.float32),
            pltpu.VMEM((CH, LW), jnp.float32),
            pltpu.VMEM((CH, LW), jnp.float32),
            pltpu.VMEM((h,), jnp.float32),
            pltpu.SemaphoreType.DMA,
            pltpu.SemaphoreType.DMA,
            pltpu.SemaphoreType.DMA,
            pltpu.SemaphoreType.DMA,
            pltpu.SemaphoreType.DMA,
        ],
    )
    def sc_fn(tab_hbm, self_hbm, idx_hbm, out_hbm, idx_v, acc_v, g0, g1,
              sum_v, s_acc, s0, s1, s_idx, s_out):
        wid = lax.axis_index("s") * 2 + lax.axis_index("c")
        base0 = wid * PER_W
        bufs = (g0, g1)
        sems = (s0, s1)
        if fuse_mean:
            for j in range(grp):
                sum_v[pl.ds(j * 16, 16)] = jnp.zeros((16,), jnp.float32)

        def load_idx(base):
            for r in range(NUM_REL):
                pltpu.async_copy(
                    idx_hbm.at[pl.ds(r * NP + base, CH)], idx_v[r], s_idx)

        load_idx(base0)

        def wait_idx():
            for r in range(NUM_REL):
                pltpu.make_async_copy(
                    idx_hbm.at[pl.ds(0, CH)], idx_v[r], s_idx).wait()

        def chunk_body(c, carry):
            base = base0 + c * CH
            wait_idx()
            handles = [
                pltpu.async_copy(tab_hbm.at[0].at[idx_v[0]], g0, s0),
                pltpu.async_copy(tab_hbm.at[1 // k].at[idx_v[1]], g1, s1),
            ]

            if not fuse_mean:
                @pl.when(c != 0)
                def _():
                    pltpu.make_async_copy(
                        acc_v, out_hbm.at[pl.ds(base, CH)], s_out).wait()
            acc_cp = pltpu.async_copy(
                self_hbm.at[pl.ds(base, CH)], acc_v, s_acc)
            acc_cp.wait()
            cnext = jnp.where(c + 1 == NCH, 0, c + 1)
            for r in range(NUM_REL):
                p = r % 2
                buf = bufs[p]
                handles[p].wait()
                last = r == NUM_REL - 1
                off = (r % k) * h
                if last:
                    load_idx(base0 + cnext * CH)

                def acc_body(i, _, buf=buf, last=last, off=off):
                    row = i * unr
                    for u in range(unr):
                        for j in range(grp):
                            dst = pl.ds(j * 16, 16)
                            src = pl.ds(off + j * 16, 16)
                            v = acc_v[row + u, dst] + buf[row + u, src]
                            if last:
                                v = jnp.maximum(v, 0.0)
                                if fuse_mean:
                                    live = base + row + u < N
                                    sum_v[dst] += jnp.where(live, v, 0.0)
                                else:
                                    acc_v[row + u, dst] = v
                            else:
                                acc_v[row + u, dst] = v
                    return 0

                lax.fori_loop(0, CH // unr, acc_body, 0)
                if r + 2 < NUM_REL:
                    handles[p] = pltpu.async_copy(
                        tab_hbm.at[(r + 2) // k].at[idx_v[r + 2]], buf,
                        sems[p])
            if not fuse_mean:
                pltpu.async_copy(acc_v, out_hbm.at[pl.ds(base, CH)], s_out)
            return carry

        lax.fori_loop(0, NCH, chunk_body, 0)
        wait_idx()
        if fuse_mean:
            pltpu.sync_copy(sum_v, out_hbm.at[wid])
        else:
            pltpu.make_async_copy(
                acc_v, out_hbm.at[pl.ds(base0, CH)], s_out).wait()

    return sc_fn


_sc_gather_h1 = _make_sc_gather(H1)
_sc_gather_h2 = _make_sc_gather(H2, fuse_mean=True)


def _tc_final_body(sums_ref, wf_ref, bf_ref, o_ref):
    mean = jnp.sum(sums_ref[...], axis=0, keepdims=True) * (1.0 / N)
    o_ref[...] = jax.lax.dot_general(
        mean, wf_ref[...], (((1,), (1,)), ((), ())),
        preferred_element_type=jnp.float32) + bf_ref[...]


def _tc_final(sums, wf, bf):
    return pl.pallas_call(
        _tc_final_body,
        out_shape=jax.ShapeDtypeStruct((1, D), jnp.float32),
    )(sums, wf, bf[None])


def kernel(node_features, adjacency_list, W1, b1, R1, W2, b2, R2, Wf, bf):
    pad = NP - N
    feats = jnp.pad(node_features, ((0, pad), (0, 0)))
    pad_idx = jnp.broadcast_to(
        jnp.arange(N, NP, dtype=jnp.int32), (NUM_REL, pad))
    adjp = jnp.concatenate([adjacency_list, pad_idx], axis=1)
    idx = adjp.reshape(NUM_REL * NP)

    wbig1 = jnp.concatenate([W1.T] + [R1[r].T for r in range(NUM_REL)], axis=1)
    bbig1 = jnp.concatenate(
        [b1, jnp.zeros((NUM_REL * H1,), jnp.float32)])[None]
    wbig2 = jnp.concatenate([W2.T] + [R2[r].T for r in range(NUM_REL)], axis=1)
    bbig2 = jnp.concatenate(
        [b2, jnp.zeros((NUM_REL * H2,), jnp.float32)])[None]

    t1, s1 = _tc_slab(feats, wbig1, bbig1, H1, blk=1792)
    h1 = _sc_gather_h1(t1, s1, idx)
    t2, s2 = _tc_slab(h1, wbig2, bbig2, H2, blk=1792)
    sums = _sc_gather_h2(t2, s2, idx)
    out = _tc_final(sums, Wf, bf)
    return out[0]
